# baseline (speedup 1.0000x reference)
import functools

import jax
import jax.numpy as jnp
from jax import lax
from jax.experimental import pallas as pl
import jax.experimental.pallas.tpu as pltpu

B, T = 16, 576
KEY_DIM = 256
N_E = 1024
E_DIM = 256
E_SPLIT = 4
KT = 0.1
EPS = 1e-12


def _ones_block(n, chunk):
    row = lax.broadcasted_iota(jnp.int32, (n, n), 0) // chunk
    col = lax.broadcasted_iota(jnp.int32, (n, n), 1) // chunk
    return (row == col).astype(jnp.float32)


BB = 4
RT = BB * T


def _fused_body(x_ref, keys_ref, r_ref, vp_ref,
                vs_ref, vh_ref, idx_ref, score_ref,
                ks_s, vpn_s):
    @pl.when(pl.program_id(0) == 0)
    def _prep():
        k = keys_ref[...]
        kn = jnp.sqrt(jnp.sum(k * k, axis=1, keepdims=True))
        r = jnp.clip(r_ref[...], 0.0, 1.0)
        ks_s[...] = k * (r / jnp.maximum(kn, EPS))
        v = vp_ref[...]
        ones_chunk = _ones_block(E_DIM, E_DIM // E_SPLIT)
        vss = lax.dot_general(v * v, ones_chunk, (((1,), (0,)), ((), ())),
                              preferred_element_type=jnp.float32)
        vpn_s[...] = v / jnp.maximum(jnp.sqrt(vss), EPS)

    x = x_ref[...].reshape(RT, KEY_DIM)
    xn = jnp.sqrt(jnp.sum(x * x, axis=1, keepdims=True))
    x = x / jnp.maximum(xn, EPS)

    score = lax.dot_general(x, ks_s[...], (((1,), (1,)), ((), ())),
                            preferred_element_type=jnp.float32)
    score_ref[...] = score.reshape(BB, T, N_E)

    m = jnp.max(score, axis=1, keepdims=True)
    iota = lax.broadcasted_iota(jnp.int32, (RT, N_E), 1)
    idx = jnp.min(jnp.where(score == m, iota, N_E), axis=1)
    idx_ref[...] = idx.reshape(BB, 1, T)

    e = jnp.exp((score * (1.0 / KT)).astype(jnp.bfloat16))
    s = jnp.sum(e.astype(jnp.float32), axis=1, keepdims=True)

    vpn = vpn_s[...]
    acc = lax.dot_general(e, vpn, (((1,), (0,)), ((), ())),
                          preferred_element_type=jnp.float32)
    vs_ref[...] = (acc / s).reshape(BB, T, E_DIM)

    onehot = (iota == idx[:, None]).astype(jnp.float32)
    vh_ref[...] = lax.dot_general(onehot, vpn, (((1,), (0,)), ((), ())),
                                  preferred_element_type=jnp.float32
                                  ).reshape(BB, T, E_DIM)


@functools.partial(jax.jit, static_argnames=("interpret",))
def _run(key_soft, keys_w, r_keys_w, vparams_w, interpret=False):
    out_shapes = (
        jax.ShapeDtypeStruct((B, T, E_DIM), jnp.float32),
        jax.ShapeDtypeStruct((B, T, E_DIM), jnp.float32),
        jax.ShapeDtypeStruct((B, 1, T), jnp.int32),
        jax.ShapeDtypeStruct((B, T, N_E), jnp.float32),
    )
    in_specs = [
        pl.BlockSpec((BB, T, KEY_DIM), lambda i: (i, 0, 0)),
        pl.BlockSpec((N_E, KEY_DIM), lambda i: (0, 0)),
        pl.BlockSpec((N_E, 1), lambda i: (0, 0)),
        pl.BlockSpec((N_E, E_DIM), lambda i: (0, 0)),
    ]
    out_specs = (
        pl.BlockSpec((BB, T, E_DIM), lambda i: (i, 0, 0)),
        pl.BlockSpec((BB, T, E_DIM), lambda i: (i, 0, 0)),
        pl.BlockSpec((BB, 1, T), lambda i: (i, 0, 0)),
        pl.BlockSpec((BB, T, N_E), lambda i: (i, 0, 0)),
    )
    return pl.pallas_call(
        _fused_body,
        grid=(B // BB,),
        in_specs=in_specs,
        out_specs=out_specs,
        out_shape=out_shapes,
        scratch_shapes=[
            pltpu.VMEM((N_E, KEY_DIM), jnp.float32),
            pltpu.VMEM((N_E, E_DIM), jnp.float32),
        ],
        interpret=interpret,
    )(key_soft, keys_w, r_keys_w, vparams_w)


def kernel(key_soft, u_t, keys_w, r_keys_w, vparams_w):
    v_soft, v_hard, idx, score = _run(key_soft, keys_w, r_keys_w, vparams_w)
    return v_soft, v_hard, idx.reshape(B, T), score


# speedup vs baseline: 1.0211x; 1.0211x over previous
import functools

import jax
import jax.numpy as jnp
from jax import lax
from jax.experimental import pallas as pl
import jax.experimental.pallas.tpu as pltpu

B, T = 16, 576
KEY_DIM = 256
N_E = 1024
E_DIM = 256
E_SPLIT = 4
KT = 0.1
EPS = 1e-12


def _ones_block(n, chunk):
    row = lax.broadcasted_iota(jnp.int32, (n, n), 0) // chunk
    col = lax.broadcasted_iota(jnp.int32, (n, n), 1) // chunk
    return (row == col).astype(jnp.float32)


BB = 4
RT = BB * T


def _fused_body(x_ref, keys_ref, r_ref, vp_ref,
                vs_ref, vh_ref, idx_ref, score_ref,
                ks_s, vpn_s):
    @pl.when(pl.program_id(0) == 0)
    def _prep():
        k = keys_ref[...]
        kn = jnp.sqrt(jnp.sum(k * k, axis=1, keepdims=True))
        r = jnp.clip(r_ref[...], 0.0, 1.0)
        ks_s[...] = k * (r / jnp.maximum(kn, EPS))
        v = vp_ref[...]
        ones_chunk = _ones_block(E_DIM, E_DIM // E_SPLIT)
        vss = lax.dot_general(v * v, ones_chunk, (((1,), (0,)), ((), ())),
                              preferred_element_type=jnp.float32)
        vpn_s[...] = v / jnp.maximum(jnp.sqrt(vss), EPS)

    x = x_ref[...].reshape(RT, KEY_DIM)
    xn = jnp.sqrt(jnp.sum(x * x, axis=1, keepdims=True))
    x = x / jnp.maximum(xn, EPS)

    score = lax.dot_general(x, ks_s[...], (((1,), (1,)), ((), ())),
                            preferred_element_type=jnp.float32)
    score_ref[...] = score.reshape(BB, T, N_E)

    m = jnp.max(score, axis=1, keepdims=True)
    iota = lax.broadcasted_iota(jnp.int32, (RT, N_E), 1)
    idx = jnp.min(jnp.where(score == m, iota, N_E), axis=1)
    idx_ref[...] = idx.reshape(BB, 1, T)

    e = jnp.exp(score * (1.0 / KT))
    s = jnp.sum(e, axis=1, keepdims=True)

    vpn = vpn_s[...]
    acc = lax.dot_general(e, vpn, (((1,), (0,)), ((), ())),
                          preferred_element_type=jnp.float32)
    vs_ref[...] = (acc / s).reshape(BB, T, E_DIM)

    onehot = (iota == idx[:, None]).astype(jnp.float32)
    vh_ref[...] = lax.dot_general(onehot, vpn, (((1,), (0,)), ((), ())),
                                  preferred_element_type=jnp.float32
                                  ).reshape(BB, T, E_DIM)


@functools.partial(jax.jit, static_argnames=("interpret",))
def _run(key_soft, keys_w, r_keys_w, vparams_w, interpret=False):
    out_shapes = (
        jax.ShapeDtypeStruct((B, T, E_DIM), jnp.float32),
        jax.ShapeDtypeStruct((B, T, E_DIM), jnp.float32),
        jax.ShapeDtypeStruct((B, 1, T), jnp.int32),
        jax.ShapeDtypeStruct((B, T, N_E), jnp.float32),
    )
    in_specs = [
        pl.BlockSpec((BB, T, KEY_DIM), lambda i: (i, 0, 0)),
        pl.BlockSpec((N_E, KEY_DIM), lambda i: (0, 0)),
        pl.BlockSpec((N_E, 1), lambda i: (0, 0)),
        pl.BlockSpec((N_E, E_DIM), lambda i: (0, 0)),
    ]
    out_specs = (
        pl.BlockSpec((BB, T, E_DIM), lambda i: (i, 0, 0)),
        pl.BlockSpec((BB, T, E_DIM), lambda i: (i, 0, 0)),
        pl.BlockSpec((BB, 1, T), lambda i: (i, 0, 0)),
        pl.BlockSpec((BB, T, N_E), lambda i: (i, 0, 0)),
    )
    return pl.pallas_call(
        _fused_body,
        grid=(B // BB,),
        in_specs=in_specs,
        out_specs=out_specs,
        out_shape=out_shapes,
        scratch_shapes=[
            pltpu.VMEM((N_E, KEY_DIM), jnp.float32),
            pltpu.VMEM((N_E, E_DIM), jnp.float32),
        ],
        interpret=interpret,
    )(key_soft, keys_w, r_keys_w, vparams_w)


def kernel(key_soft, u_t, keys_w, r_keys_w, vparams_w):
    v_soft, v_hard, idx, score = _run(key_soft, keys_w, r_keys_w, vparams_w)
    return v_soft, v_hard, idx.reshape(B, T), score
